# baseline (device time: 12863 ns/iter reference)
import jax
import jax.numpy as jnp
from jax import lax
from jax.experimental import pallas as pl
from jax.experimental.pallas import tpu as pltpu

N_DEV = 4
BM = 256


def kernel(x):
    m, n = x.shape
    G = m // BM
    assert m % BM == 0

    def body(x_blk, x_hbm, out_blk, top_halo, bot_halo, prev_row, next_row,
             send_sems, recv_sems, local_sems):
        k = pl.program_id(0)
        b = lax.rem(k + 1, G)
        my_pos = lax.axis_index("i")
        left = (my_pos - 1) % N_DEV
        right = (my_pos + 1) % N_DEV

        send_right = pltpu.make_async_remote_copy(
            src_ref=x_hbm.at[pl.ds(m - 1, 1)],
            dst_ref=top_halo,
            send_sem=send_sems.at[0],
            recv_sem=recv_sems.at[0],
            device_id=(right,),
            device_id_type=pl.DeviceIdType.MESH,
        )
        send_left = pltpu.make_async_remote_copy(
            src_ref=x_hbm.at[pl.ds(0, 1)],
            dst_ref=bot_halo,
            send_sem=send_sems.at[1],
            recv_sem=recv_sems.at[1],
            device_id=(left,),
            device_id_type=pl.DeviceIdType.MESH,
        )

        @pl.when(k == 0)
        def _():
            barrier_sem = pltpu.get_barrier_semaphore()
            for nbr in [left, right]:
                pl.semaphore_signal(
                    barrier_sem, inc=1,
                    device_id=(nbr,), device_id_type=pl.DeviceIdType.MESH,
                )
            pl.semaphore_wait(barrier_sem, 2)
            send_right.start()
            send_left.start()

        prev_start = jnp.maximum(b * BM - 1, 0)
        next_start = jnp.minimum((b + 1) * BM, m - 1)
        copy_prev = pltpu.make_async_copy(
            x_hbm.at[pl.ds(prev_start, 1)], prev_row, local_sems.at[0])
        copy_next = pltpu.make_async_copy(
            x_hbm.at[pl.ds(next_start, 1)], next_row, local_sems.at[1])
        copy_prev.start()
        copy_next.start()

        out_blk[pl.ds(1, BM - 2), :] = (
            0.25 * x_blk[pl.ds(0, BM - 2), :]
            + 0.5 * x_blk[pl.ds(1, BM - 2), :]
            + 0.25 * x_blk[pl.ds(2, BM - 2), :]
        )

        copy_prev.wait()
        copy_next.wait()

        @pl.when(b != 0)
        def _():
            out_blk[pl.ds(0, 1), :] = (
                0.25 * prev_row[:, :]
                + 0.5 * x_blk[pl.ds(0, 1), :]
                + 0.25 * x_blk[pl.ds(1, 1), :]
            )

        @pl.when(b != G - 1)
        def _():
            out_blk[pl.ds(BM - 1, 1), :] = (
                0.25 * x_blk[pl.ds(BM - 2, 1), :]
                + 0.5 * x_blk[pl.ds(BM - 1, 1), :]
                + 0.25 * next_row[:, :]
            )

        @pl.when(b == G - 1)
        def _():
            send_left.wait_recv()

            @pl.when(my_pos == N_DEV - 1)
            def _():
                out_blk[pl.ds(BM - 1, 1), :] = x_blk[pl.ds(BM - 1, 1), :]

            @pl.when(my_pos != N_DEV - 1)
            def _():
                out_blk[pl.ds(BM - 1, 1), :] = (
                    0.25 * x_blk[pl.ds(BM - 2, 1), :]
                    + 0.5 * x_blk[pl.ds(BM - 1, 1), :]
                    + 0.25 * bot_halo[:, :]
                )

        @pl.when(b == 0)
        def _():
            send_right.wait_recv()
            send_right.wait_send()
            send_left.wait_send()

            @pl.when(my_pos == 0)
            def _():
                out_blk[pl.ds(0, 1), :] = x_blk[pl.ds(0, 1), :]

            @pl.when(my_pos != 0)
            def _():
                out_blk[pl.ds(0, 1), :] = (
                    0.25 * top_halo[:, :]
                    + 0.5 * x_blk[pl.ds(0, 1), :]
                    + 0.25 * x_blk[pl.ds(1, 1), :]
                )

    return pl.pallas_call(
        body,
        grid=(G,),
        out_shape=jax.ShapeDtypeStruct((m, n), x.dtype),
        in_specs=[
            pl.BlockSpec((BM, n), lambda k: ((k + 1) % G, 0)),
            pl.BlockSpec(memory_space=pl.ANY),
        ],
        out_specs=pl.BlockSpec((BM, n), lambda k: ((k + 1) % G, 0)),
        scratch_shapes=[
            pltpu.VMEM((1, n), x.dtype),
            pltpu.VMEM((1, n), x.dtype),
            pltpu.VMEM((1, n), x.dtype),
            pltpu.VMEM((1, n), x.dtype),
            pltpu.SemaphoreType.DMA((2,)),
            pltpu.SemaphoreType.DMA((2,)),
            pltpu.SemaphoreType.DMA((2,)),
        ],
        compiler_params=pltpu.CompilerParams(collective_id=0),
    )(x, x)


# device time: 12707 ns/iter; 1.0123x vs baseline; 1.0123x over previous
import jax
import jax.numpy as jnp
from jax import lax
from jax.experimental import pallas as pl
from jax.experimental.pallas import tpu as pltpu

N_DEV = 4
NCHUNK = 16


def _halo_rows_kernel(x):
    m, n = x.shape

    def body(x_hbm, out_ref, edge_stage, top_halo, bot_halo,
             send_sems, recv_sems, stage_sems):
        my_pos = lax.axis_index("i")
        left = (my_pos - 1) % N_DEV
        right = (my_pos + 1) % N_DEV

        top_copy = pltpu.make_async_copy(
            x_hbm.at[pl.ds(0, 2)], edge_stage.at[pl.ds(0, 2)],
            stage_sems.at[0])
        bot_copy = pltpu.make_async_copy(
            x_hbm.at[pl.ds(m - 2, 2)], edge_stage.at[pl.ds(2, 2)],
            stage_sems.at[1])
        top_copy.start()
        bot_copy.start()

        barrier_sem = pltpu.get_barrier_semaphore()
        for nbr in [left, right]:
            pl.semaphore_signal(
                barrier_sem, inc=1,
                device_id=(nbr,), device_id_type=pl.DeviceIdType.MESH,
            )
        pl.semaphore_wait(barrier_sem, 2)

        send_right = pltpu.make_async_remote_copy(
            src_ref=x_hbm.at[pl.ds(m - 1, 1)],
            dst_ref=top_halo,
            send_sem=send_sems.at[0],
            recv_sem=recv_sems.at[0],
            device_id=(right,),
            device_id_type=pl.DeviceIdType.MESH,
        )
        send_left = pltpu.make_async_remote_copy(
            src_ref=x_hbm.at[pl.ds(0, 1)],
            dst_ref=bot_halo,
            send_sem=send_sems.at[1],
            recv_sem=recv_sems.at[1],
            device_id=(left,),
            device_id_type=pl.DeviceIdType.MESH,
        )
        send_right.start()
        send_left.start()

        top_copy.wait()
        bot_copy.wait()
        send_right.wait_recv()
        send_left.wait_recv()

        @pl.when(my_pos == 0)
        def _():
            out_ref[pl.ds(0, 1), :] = edge_stage[pl.ds(0, 1), :]

        @pl.when(my_pos != 0)
        def _():
            out_ref[pl.ds(0, 1), :] = (
                0.25 * top_halo[:, :]
                + 0.5 * edge_stage[pl.ds(0, 1), :]
                + 0.25 * edge_stage[pl.ds(1, 1), :]
            )

        @pl.when(my_pos == N_DEV - 1)
        def _():
            out_ref[pl.ds(1, 1), :] = edge_stage[pl.ds(3, 1), :]

        @pl.when(my_pos != N_DEV - 1)
        def _():
            out_ref[pl.ds(1, 1), :] = (
                0.25 * edge_stage[pl.ds(2, 1), :]
                + 0.5 * edge_stage[pl.ds(3, 1), :]
                + 0.25 * bot_halo[:, :]
            )

        send_right.wait_send()
        send_left.wait_send()

    return pl.pallas_call(
        body,
        out_shape=jax.ShapeDtypeStruct((2, n), x.dtype),
        in_specs=[pl.BlockSpec(memory_space=pltpu.MemorySpace.HBM)],
        out_specs=pl.BlockSpec(memory_space=pltpu.MemorySpace.VMEM),
        scratch_shapes=[
            pltpu.VMEM((4, n), x.dtype),
            pltpu.VMEM((1, n), x.dtype),
            pltpu.VMEM((1, n), x.dtype),
            pltpu.SemaphoreType.DMA((2,)),
            pltpu.SemaphoreType.DMA((2,)),
            pltpu.SemaphoreType.DMA((2,)),
        ],
        compiler_params=pltpu.CompilerParams(collective_id=0),
    )(pltpu.with_memory_space_constraint(x, pltpu.MemorySpace.HBM))


def _stencil_kernel(x, boundary):
    m, n = x.shape
    C = m // NCHUNK
    assert m % NCHUNK == 0

    def body(x_hbm, bnd_ref, out_hbm, in_stage, out_stage, in_sems, out_sems):
        def in_copy(c):
            return pltpu.make_async_copy(
                x_hbm.at[pl.ds(c * C, C)], in_stage.at[c], in_sems.at[c])

        def out_copy(c):
            return pltpu.make_async_copy(
                out_stage.at[c], out_hbm.at[pl.ds(c * C, C)], out_sems.at[c])

        for c in range(NCHUNK):
            in_copy(c).start()

        for c in range(NCHUNK):
            if c == 0:
                in_copy(0).wait()
                in_copy(1).wait()
            elif c + 1 < NCHUNK:
                in_copy(c + 1).wait()

            out_stage[c, pl.ds(1, C - 2), :] = (
                0.25 * in_stage[c, pl.ds(0, C - 2), :]
                + 0.5 * in_stage[c, pl.ds(1, C - 2), :]
                + 0.25 * in_stage[c, pl.ds(2, C - 2), :]
            )

            if c > 0:
                out_stage[c, pl.ds(0, 1), :] = (
                    0.25 * in_stage[c - 1, pl.ds(C - 1, 1), :]
                    + 0.5 * in_stage[c, pl.ds(0, 1), :]
                    + 0.25 * in_stage[c, pl.ds(1, 1), :]
                )
            else:
                out_stage[0, pl.ds(0, 1), :] = bnd_ref[pl.ds(0, 1), :]

            if c < NCHUNK - 1:
                out_stage[c, pl.ds(C - 1, 1), :] = (
                    0.25 * in_stage[c, pl.ds(C - 2, 1), :]
                    + 0.5 * in_stage[c, pl.ds(C - 1, 1), :]
                    + 0.25 * in_stage[c + 1, pl.ds(0, 1), :]
                )
            else:
                out_stage[c, pl.ds(C - 1, 1), :] = bnd_ref[pl.ds(1, 1), :]

            out_copy(c).start()

        for c in range(NCHUNK):
            out_copy(c).wait()

    return pl.pallas_call(
        body,
        out_shape=jax.ShapeDtypeStruct((m, n), x.dtype),
        in_specs=[
            pl.BlockSpec(memory_space=pltpu.MemorySpace.HBM),
            pl.BlockSpec(memory_space=pltpu.MemorySpace.VMEM),
        ],
        out_specs=pl.BlockSpec(memory_space=pltpu.MemorySpace.HBM),
        scratch_shapes=[
            pltpu.VMEM((NCHUNK, C, n), x.dtype),
            pltpu.VMEM((NCHUNK, C, n), x.dtype),
            pltpu.SemaphoreType.DMA((NCHUNK,)),
            pltpu.SemaphoreType.DMA((NCHUNK,)),
        ],
        compiler_params=pltpu.CompilerParams(
            vmem_limit_bytes=100 * 1024 * 1024,
        ),
    )(
        pltpu.with_memory_space_constraint(x, pltpu.MemorySpace.HBM),
        boundary,
    )


def kernel(x):
    boundary = _halo_rows_kernel(x)
    return _stencil_kernel(x, boundary)
